# baseline (speedup 1.0000x reference)
import functools
import jax
import jax.numpy as jnp
from jax import lax
from jax.experimental import pallas as pl
from jax.experimental.pallas import tpu as pltpu, tpu_sc as plsc


def _make_sc_kernel(b, d, h, w):
    mesh = plsc.VectorSubcoreMesh(core_axis_name="c", subcore_axis_name="s")
    hw = h * w

    @functools.partial(
        pl.kernel,
        out_type=jax.ShapeDtypeStruct((b, hw, 2 * d), jnp.float32),
        mesh=mesh,
        scratch_types=[
            pltpu.VMEM((2 * h, d), jnp.float32),
            pltpu.VMEM((w, 2 * d), jnp.float32),
            pltpu.SemaphoreType.DMA,
        ],
    )
    def k(row_hbm, col_hbm, out_hbm, t, slab, sem):
        nc = 2
        wid = lax.axis_index("s") * nc + lax.axis_index("c")

        pltpu.sync_copy(col_hbm.at[pl.ds(0, h)], t.at[pl.ds(0, h)])
        pltpu.sync_copy(row_hbm.at[pl.ds(0, h)], t.at[pl.ds(h, h)])

        nv = d // 16
        rrow = h + wid
        rvecs = [t[rrow, pl.ds(16 * k_, 16)] for k_ in range(nv)]
        for j in range(w):
            for k_ in range(nv):
                slab[j, pl.ds(16 * k_, 16)] = t[j, pl.ds(16 * k_, 16)]
                slab[j, pl.ds(d + 16 * k_, 16)] = rvecs[k_]

        r0 = pl.multiple_of(wid * w, w)
        descs = [
            pltpu.async_copy(slab, out_hbm.at[bi, pl.ds(r0, w)], sem)
            for bi in range(b)
        ]
        for de in descs:
            de.wait()

    return k


def kernel(x, row_embed, col_embed):
    b = x.shape[0]
    h, w = x.shape[-2], x.shape[-1]
    d = col_embed.shape[-1]
    out = _make_sc_kernel(b, d, h, w)(row_embed, col_embed)
    return out.reshape(b, h, w, 2 * d).transpose(0, 3, 1, 2)


# speedup vs baseline: 2.9498x; 2.9498x over previous
import jax
import jax.numpy as jnp
from jax.experimental import pallas as pl
from jax.experimental.pallas import tpu as pltpu


def _make_body(b, d, h, w):
    hw = h * w

    def body(row_ref, col_ref, out_ref, scratch, sems):
        col = col_ref[0:w, :]
        row = row_ref[0:h, :]
        xp = jnp.broadcast_to(col[None, :, :], (h, w, d)).reshape(hw, d)
        yp = jnp.broadcast_to(row[:, None, :], (h, w, d)).reshape(hw, d)
        scratch[:, 0:d] = xp
        scratch[:, d:2 * d] = yp
        half = hw // 2
        copies = [
            pltpu.make_async_copy(
                scratch.at[pl.ds(s * half, half)],
                out_ref.at[i, pl.ds(s * half, half)],
                sems.at[2 * i + s],
            )
            for i in range(b)
            for s in range(2)
        ]
        for c in copies:
            c.start()
        for c in copies:
            c.wait()
    return body


def kernel(x, row_embed, col_embed):
    b = x.shape[0]
    h, w = x.shape[-2], x.shape[-1]
    d = col_embed.shape[-1]
    hw = h * w
    out_phys = pl.pallas_call(
        _make_body(b, d, h, w),
        in_specs=[
            pl.BlockSpec(memory_space=pltpu.VMEM),
            pl.BlockSpec(memory_space=pltpu.VMEM),
        ],
        out_specs=pl.BlockSpec(memory_space=pltpu.MemorySpace.HBM),
        out_shape=jax.ShapeDtypeStruct((b, hw, 2 * d), jnp.float32),
        scratch_shapes=[
            pltpu.VMEM((hw, 2 * d), jnp.float32),
            pltpu.SemaphoreType.DMA((2 * b,)),
        ],
    )(row_embed, col_embed)
    return out_phys.reshape(b, h, w, 2 * d).transpose(0, 3, 1, 2)


# speedup vs baseline: 2.9558x; 1.0020x over previous
import jax
import jax.numpy as jnp
from jax.experimental import pallas as pl
from jax.experimental.pallas import tpu as pltpu


def _make_body(b, d, h, w):
    hw = h * w

    def body(row_ref, col_ref, out_ref, scratch, sems):
        col = col_ref[0:w, :]
        row = row_ref[0:h, :]
        h2 = h // 2
        copies = []
        for s in range(2):
            rows = row[s * h2:(s + 1) * h2, :]
            xp = jnp.broadcast_to(col[None, :, :], (h2, w, d))
            yp = jnp.broadcast_to(rows[:, None, :], (h2, w, d))
            lo = s * (hw // 2)
            scratch[pl.ds(lo, hw // 2), 0:d] = xp.reshape(hw // 2, d)
            scratch[pl.ds(lo, hw // 2), d:2 * d] = yp.reshape(hw // 2, d)
            for i in range(b):
                c = pltpu.make_async_copy(
                    scratch.at[pl.ds(lo, hw // 2)],
                    out_ref.at[i, pl.ds(lo, hw // 2)],
                    sems.at[2 * i + s],
                )
                c.start()
                copies.append(c)
        for c in copies:
            c.wait()
    return body


def kernel(x, row_embed, col_embed):
    b = x.shape[0]
    h, w = x.shape[-2], x.shape[-1]
    d = col_embed.shape[-1]
    hw = h * w
    out_phys = pl.pallas_call(
        _make_body(b, d, h, w),
        in_specs=[
            pl.BlockSpec(memory_space=pltpu.VMEM),
            pl.BlockSpec(memory_space=pltpu.VMEM),
        ],
        out_specs=pl.BlockSpec(memory_space=pltpu.MemorySpace.HBM),
        out_shape=jax.ShapeDtypeStruct((b, hw, 2 * d), jnp.float32),
        scratch_shapes=[
            pltpu.VMEM((hw, 2 * d), jnp.float32),
            pltpu.SemaphoreType.DMA((2 * b,)),
        ],
    )(row_embed, col_embed)
    return out_phys.reshape(b, h, w, 2 * d).transpose(0, 3, 1, 2)


# speedup vs baseline: 3.0187x; 1.0213x over previous
import jax
import jax.numpy as jnp
from jax.experimental import pallas as pl
from jax.experimental.pallas import tpu as pltpu


def _make_body(b, d, h, w):
    hw = h * w

    def body(row_ref, col_ref, out_ref, scratch, sems):
        col = col_ref[0:w, :]
        row = row_ref[0:h, :]
        xp = jnp.broadcast_to(col[None, :, :], (h, w, d)).reshape(hw, d)
        yp = jnp.broadcast_to(row[:, None, :], (h, w, d)).reshape(hw, d)
        scratch[:, 0:d] = xp
        scratch[:, d:2 * d] = yp
        copies = [
            pltpu.make_async_copy(scratch, out_ref.at[i], sems.at[i])
            for i in range(b)
        ]
        for c in copies:
            c.start()
        for c in copies:
            c.wait()
    return body


def kernel(x, row_embed, col_embed):
    b = x.shape[0]
    h, w = x.shape[-2], x.shape[-1]
    d = col_embed.shape[-1]
    hw = h * w
    out_phys = pl.pallas_call(
        _make_body(b, d, h, w),
        in_specs=[
            pl.BlockSpec(memory_space=pltpu.VMEM),
            pl.BlockSpec(memory_space=pltpu.VMEM),
        ],
        out_specs=pl.BlockSpec(memory_space=pltpu.MemorySpace.HBM),
        out_shape=jax.ShapeDtypeStruct((b, hw, 2 * d), jnp.float32),
        scratch_shapes=[
            pltpu.VMEM((hw, 2 * d), jnp.float32),
            pltpu.SemaphoreType.DMA((b,)),
        ],
    )(row_embed, col_embed)
    return out_phys.reshape(b, h, w, 2 * d).transpose(0, 3, 1, 2)
